# baseline (speedup 1.0000x reference)
import functools

import jax
import jax.numpy as jnp
from jax import lax
from jax.experimental import pallas as pl
from jax.experimental.pallas import tpu as pltpu
from jax.experimental.pallas import tpu_sc as plsc

N = 10000
NP = 10240
E = 320000
D = 128
H = 16
G = 64
O = 2

ECH = 128
NC = 2
NS = 16
NB = 16
NCH = 2560
KPW0 = 128
KPW1 = 32
CORE1_BASE = NS * KPW0
KPWMAX = max(KPW0, KPW1)
NCHA = CORE1_BASE + (NS - 1) * KPW1 + KPWMAX
EP = NCHA * ECH
RPS = NP // NS

BLK = 2048
NBLK = NP // BLK

_MESH = dict(core_axis_name="c", subcore_axis_name="s", num_cores=NC,
             num_subcores=NS)


def _zero_fill(buf, nrows):
    def zrow(r, _):
        buf[r, :] = jnp.zeros((H,), jnp.float32)
        return 0
    lax.fori_loop(0, nrows, zrow, 0)


def _writeback(acc_sh, buf_v, out_hbm, c, s):
    pltpu.sync_copy(acc_sh.at[pl.ds(s * RPS, RPS)], buf_v)
    pltpu.sync_copy(buf_v, out_hbm.at[c, pl.ds(s * RPS, RPS)])


def _worker_chunks(c, s):
    w0 = jnp.where(c == 0, s * KPW0, CORE1_BASE + s * KPW1)
    ngrp = jnp.where(c == 0, KPW0 // NB, KPW1 // NB)
    return w0, ngrp


def _sc_deg_body(e3_hbm, out_hbm, idx_v, ones_v, buf_v, acc_sh, sem_i, sem_s):
    c = lax.axis_index("c")
    s = lax.axis_index("s")
    w0, ngrp = _worker_chunks(c, s)
    icp = pltpu.async_copy(e3_hbm.at[pl.ds(w0, KPWMAX)], idx_v, sem_i)
    _zero_fill(buf_v, RPS)
    pltpu.sync_copy(buf_v, acc_sh.at[pl.ds(s * RPS, RPS)])

    def orow(r, _):
        ones_v[r, :] = jnp.ones((H,), jnp.float32)
        return 0
    lax.fori_loop(0, ECH, orow, 0)
    icp.wait()
    plsc.subcore_barrier()

    def group(g, _):
        sps = [pltpu.async_copy(ones_v, acc_sh.at[idx_v.at[g * NB + j, 1]],
                                sem_s, add=True)
               for j in range(NB)]
        for sp in sps:
            sp.wait()
        return 0
    lax.fori_loop(0, ngrp, group, 0)

    plsc.subcore_barrier()
    _writeback(acc_sh, buf_v, out_hbm, c, s)


def _make_sc_deg(interpret=False):
    return pl.kernel(
        _sc_deg_body,
        out_type=jax.ShapeDtypeStruct((NC, NP, H), jnp.float32),
        mesh=plsc.VectorSubcoreMesh(**_MESH),
        scratch_types=[
            pltpu.VMEM((KPWMAX, 2, ECH), jnp.int32),
            pltpu.VMEM((ECH, H), jnp.float32),
            pltpu.VMEM((RPS, H), jnp.float32),
            pltpu.VMEM_SHARED((NP, H), jnp.float32),
            pltpu.SemaphoreType.DMA,
            pltpu.SemaphoreType.DMA,
        ],
        compiler_params=pltpu.CompilerParams(use_tc_tiling_on_sc=False),
        interpret=interpret,
    )


def _sc_edges_body(e3_hbm, g_hbm, out_hbm,
                   idx_v, rows_v, buf_v, acc_sh, sem_i, sem_g, sem_s):
    c = lax.axis_index("c")
    s = lax.axis_index("s")
    w0, ngrp = _worker_chunks(c, s)
    icp = pltpu.async_copy(e3_hbm.at[pl.ds(w0, KPWMAX)], idx_v, sem_i)
    _zero_fill(buf_v, RPS)
    pltpu.sync_copy(buf_v, acc_sh.at[pl.ds(s * RPS, RPS)])
    icp.wait()
    plsc.subcore_barrier()

    def group(g, _):
        gs = [pltpu.async_copy(g_hbm.at[idx_v.at[g * NB + j, 0]],
                               rows_v.at[j], sem_g)
              for j in range(NB)]
        for gp in gs:
            gp.wait()
        sps = [pltpu.async_copy(rows_v.at[j],
                                acc_sh.at[idx_v.at[g * NB + j, 1]],
                                sem_s, add=True)
               for j in range(NB)]
        for sp in sps:
            sp.wait()
        return 0
    lax.fori_loop(0, ngrp, group, 0)

    plsc.subcore_barrier()
    _writeback(acc_sh, buf_v, out_hbm, c, s)


def _make_sc_edges(interpret=False):
    return pl.kernel(
        _sc_edges_body,
        out_type=jax.ShapeDtypeStruct((NC, NP, H), jnp.float32),
        mesh=plsc.VectorSubcoreMesh(**_MESH),
        scratch_types=[
            pltpu.VMEM((KPWMAX, 2, ECH), jnp.int32),
            pltpu.VMEM((NB, ECH, H), jnp.float32),
            pltpu.VMEM((RPS, H), jnp.float32),
            pltpu.VMEM_SHARED((NP, H), jnp.float32),
            pltpu.SemaphoreType.DMA,
            pltpu.SemaphoreType.DMA,
            pltpu.SemaphoreType.DMA,
        ],
        compiler_params=pltpu.CompilerParams(use_tc_tiling_on_sc=False),
        interpret=interpret,
    )


_SC_CACHE = {}


def _sc_kernels():
    if not _SC_CACHE:
        _SC_CACHE["deg"] = _make_sc_deg()
        _SC_CACHE["edges"] = _make_sc_edges()
    return _SC_CACHE["deg"], _SC_CACHE["edges"]


def _elu(t):
    return jnp.where(t > 0.0, t, jnp.exp(jnp.minimum(t, 0.0)) - 1.0)


def _tc1_body(x_ref, w_ref, dp_ref, g_ref, dinv_ref):
    deg = dp_ref[0] + dp_ref[1] + 1.0
    dinv = lax.rsqrt(deg)
    g_ref[...] = jnp.dot(x_ref[...], w_ref[...],
                         preferred_element_type=jnp.float32) * dinv
    dinv_ref[...] = dinv


def _make_tc1(interpret=False):
    return pl.pallas_call(
        _tc1_body,
        interpret=interpret,
        grid=(NBLK,),
        in_specs=[
            pl.BlockSpec((BLK, D), lambda b: (b, 0)),
            pl.BlockSpec((D, H), lambda b: (0, 0)),
            pl.BlockSpec((NC, BLK, H), lambda b: (0, b, 0)),
        ],
        out_specs=[pl.BlockSpec((BLK, H), lambda b: (b, 0)),
                   pl.BlockSpec((BLK, H), lambda b: (b, 0))],
        out_shape=[jax.ShapeDtypeStruct((NP, H), jnp.float32),
                   jax.ShapeDtypeStruct((NP, H), jnp.float32)],
    )


def _tc2_body(a_ref, g_ref, d_ref, b_ref, w_ref, o_ref):
    h1 = _elu(d_ref[...] * (a_ref[0] + a_ref[1] + g_ref[...]) + b_ref[...])
    o_ref[...] = jnp.dot(h1, w_ref[...],
                         preferred_element_type=jnp.float32) * d_ref[...]


def _make_tc2(interpret=False):
    return pl.pallas_call(
        _tc2_body,
        interpret=interpret,
        grid=(NBLK,),
        in_specs=[
            pl.BlockSpec((NC, BLK, H), lambda b: (0, b, 0)),
            pl.BlockSpec((BLK, H), lambda b: (b, 0)),
            pl.BlockSpec((BLK, H), lambda b: (b, 0)),
            pl.BlockSpec((1, H), lambda b: (0, 0)),
            pl.BlockSpec((H, H), lambda b: (0, 0)),
        ],
        out_specs=pl.BlockSpec((BLK, H), lambda b: (b, 0)),
        out_shape=jax.ShapeDtypeStruct((NP, H), jnp.float32),
    )


def _tc3_body(a_ref, g_ref, d_ref, b_ref, i_ref, w2_ref, c2_ref, w3_ref,
              c3_ref, o_ref, acc):
    pid = pl.program_id(0)
    h2 = _elu(d_ref[...] * (a_ref[0] + a_ref[1] + g_ref[...]) + b_ref[...])
    seg = i_ref[0]
    oh = (lax.broadcasted_iota(jnp.int32, (G, BLK), 0) == seg
          ).astype(jnp.float32)
    part = jnp.dot(oh, h2, preferred_element_type=jnp.float32)

    @pl.when(pid == 0)
    def _():
        acc[...] = part

    @pl.when(pid != 0)
    def _():
        acc[...] += part

    @pl.when(pid == NBLK - 1)
    def _():
        p = acc[...]
        t1 = jnp.maximum(
            jnp.dot(p, w2_ref[...], preferred_element_type=jnp.float32)
            + c2_ref[...], 0.0)
        z = jnp.dot(t1, w3_ref[...], preferred_element_type=jnp.float32) \
            + c3_ref[...]
        o_ref[...] = 1.0 / (1.0 + jnp.exp(-z))


def _make_tc3(interpret=False):
    return pl.pallas_call(
        _tc3_body,
        interpret=interpret,
        grid=(NBLK,),
        in_specs=[
            pl.BlockSpec((NC, BLK, H), lambda b: (0, b, 0)),
            pl.BlockSpec((BLK, H), lambda b: (b, 0)),
            pl.BlockSpec((BLK, H), lambda b: (b, 0)),
            pl.BlockSpec((1, H), lambda b: (0, 0)),
            pl.BlockSpec((1, 1, BLK), lambda b: (b, 0, 0)),
            pl.BlockSpec((H, H), lambda b: (0, 0)),
            pl.BlockSpec((1, H), lambda b: (0, 0)),
            pl.BlockSpec((H, O), lambda b: (0, 0)),
            pl.BlockSpec((1, O), lambda b: (0, 0)),
        ],
        out_specs=pl.BlockSpec((G, O), lambda b: (0, 0)),
        out_shape=jax.ShapeDtypeStruct((G, O), jnp.float32),
        scratch_shapes=[pltpu.VMEM((G, H), jnp.float32)],
    )


_tc1 = _make_tc1()
_tc2 = _make_tc2()
_tc3 = _make_tc3()


def kernel(x, edge_index, i, W1, b1, W2, b2, Wd2, bd2, Wd3, bd3):
    pad = jnp.full((2, EP - E), N, dtype=edge_index.dtype)
    e3 = jnp.concatenate([edge_index, pad], axis=1) \
            .reshape(2, NCHA, ECH).transpose(1, 0, 2)
    x_p = jnp.pad(x, ((0, NP - N), (0, 0)))
    i3 = jnp.pad(i, (0, NP - N), constant_values=G).reshape(NBLK, 1, BLK)

    sc_deg, sc_edges = _sc_kernels()
    degp = sc_deg(e3)
    g1, dinv = _tc1(x_p, W1, degp)
    agg1 = sc_edges(e3, g1)
    g2 = _tc2(agg1, g1, dinv, b1.reshape(1, H), W2)
    agg2 = sc_edges(e3, g2)
    out = _tc3(agg2, g2, dinv, b2.reshape(1, H), i3, Wd2,
               bd2.reshape(1, H), Wd3, bd3.reshape(1, O))
    return out


# speedup vs baseline: 1.0232x; 1.0232x over previous
import functools

import jax
import jax.numpy as jnp
from jax import lax
from jax.experimental import pallas as pl
from jax.experimental.pallas import tpu as pltpu
from jax.experimental.pallas import tpu_sc as plsc

N = 10000
NP = 10240
E = 320000
D = 128
H = 16
G = 64
O = 2

ECH = 128
NC = 2
NS = 16
NB = 16
NCH = 2560
KPW0 = 112
KPW1 = 48
CORE1_BASE = NS * KPW0
KPWMAX = max(KPW0, KPW1)
NCHA = CORE1_BASE + (NS - 1) * KPW1 + KPWMAX
EP = NCHA * ECH
RPS = NP // NS

BLK = 2048
NBLK = NP // BLK

_MESH = dict(core_axis_name="c", subcore_axis_name="s", num_cores=NC,
             num_subcores=NS)


def _zero_fill(buf, nrows):
    def zrow(r, _):
        buf[r, :] = jnp.zeros((H,), jnp.float32)
        return 0
    lax.fori_loop(0, nrows, zrow, 0)


def _writeback(acc_sh, buf_v, out_hbm, c, s):
    pltpu.sync_copy(acc_sh.at[pl.ds(s * RPS, RPS)], buf_v)
    pltpu.sync_copy(buf_v, out_hbm.at[c, pl.ds(s * RPS, RPS)])


def _worker_chunks(c, s):
    w0 = jnp.where(c == 0, s * KPW0, CORE1_BASE + s * KPW1)
    ngrp = jnp.where(c == 0, KPW0 // NB, KPW1 // NB)
    return w0, ngrp


def _sc_deg_body(e3_hbm, out_hbm, idx_v, ones_v, buf_v, acc_sh, sem_i, sem_s):
    c = lax.axis_index("c")
    s = lax.axis_index("s")
    w0, ngrp = _worker_chunks(c, s)
    icp = pltpu.async_copy(e3_hbm.at[pl.ds(w0, KPWMAX)], idx_v, sem_i)
    _zero_fill(buf_v, RPS)
    pltpu.sync_copy(buf_v, acc_sh.at[pl.ds(s * RPS, RPS)])

    def orow(r, _):
        ones_v[r, :] = jnp.ones((H,), jnp.float32)
        return 0
    lax.fori_loop(0, ECH, orow, 0)
    icp.wait()
    plsc.subcore_barrier()

    def group(g, _):
        sps = [pltpu.async_copy(ones_v, acc_sh.at[idx_v.at[g * NB + j, 1]],
                                sem_s, add=True)
               for j in range(NB)]
        for sp in sps:
            sp.wait()
        return 0
    lax.fori_loop(0, ngrp, group, 0)

    plsc.subcore_barrier()
    _writeback(acc_sh, buf_v, out_hbm, c, s)


def _make_sc_deg(interpret=False):
    return pl.kernel(
        _sc_deg_body,
        out_type=jax.ShapeDtypeStruct((NC, NP, H), jnp.float32),
        mesh=plsc.VectorSubcoreMesh(**_MESH),
        scratch_types=[
            pltpu.VMEM((KPWMAX, 2, ECH), jnp.int32),
            pltpu.VMEM((ECH, H), jnp.float32),
            pltpu.VMEM((RPS, H), jnp.float32),
            pltpu.VMEM_SHARED((NP, H), jnp.float32),
            pltpu.SemaphoreType.DMA,
            pltpu.SemaphoreType.DMA,
        ],
        compiler_params=pltpu.CompilerParams(use_tc_tiling_on_sc=False),
        interpret=interpret,
    )


def _sc_edges_body(e3_hbm, g_hbm, out_hbm,
                   idx_v, rows_v, buf_v, acc_sh, sem_i, sem_g, sem_s):
    c = lax.axis_index("c")
    s = lax.axis_index("s")
    w0, ngrp = _worker_chunks(c, s)
    icp = pltpu.async_copy(e3_hbm.at[pl.ds(w0, KPWMAX)], idx_v, sem_i)
    _zero_fill(buf_v, RPS)
    pltpu.sync_copy(buf_v, acc_sh.at[pl.ds(s * RPS, RPS)])
    icp.wait()
    plsc.subcore_barrier()

    def group(g, _):
        gs = [pltpu.async_copy(g_hbm.at[idx_v.at[g * NB + j, 0]],
                               rows_v.at[j], sem_g)
              for j in range(NB)]
        for gp in gs:
            gp.wait()
        sps = [pltpu.async_copy(rows_v.at[j],
                                acc_sh.at[idx_v.at[g * NB + j, 1]],
                                sem_s, add=True)
               for j in range(NB)]
        for sp in sps:
            sp.wait()
        return 0
    lax.fori_loop(0, ngrp, group, 0)

    plsc.subcore_barrier()
    _writeback(acc_sh, buf_v, out_hbm, c, s)


def _make_sc_edges(interpret=False):
    return pl.kernel(
        _sc_edges_body,
        out_type=jax.ShapeDtypeStruct((NC, NP, H), jnp.float32),
        mesh=plsc.VectorSubcoreMesh(**_MESH),
        scratch_types=[
            pltpu.VMEM((KPWMAX, 2, ECH), jnp.int32),
            pltpu.VMEM((NB, ECH, H), jnp.float32),
            pltpu.VMEM((RPS, H), jnp.float32),
            pltpu.VMEM_SHARED((NP, H), jnp.float32),
            pltpu.SemaphoreType.DMA,
            pltpu.SemaphoreType.DMA,
            pltpu.SemaphoreType.DMA,
        ],
        compiler_params=pltpu.CompilerParams(use_tc_tiling_on_sc=False),
        interpret=interpret,
    )


_SC_CACHE = {}


def _sc_kernels():
    if not _SC_CACHE:
        _SC_CACHE["deg"] = _make_sc_deg()
        _SC_CACHE["edges"] = _make_sc_edges()
    return _SC_CACHE["deg"], _SC_CACHE["edges"]


def _elu(t):
    return jnp.where(t > 0.0, t, jnp.exp(jnp.minimum(t, 0.0)) - 1.0)


def _tc1_body(x_ref, w_ref, dp_ref, g_ref, dinv_ref):
    deg = dp_ref[0] + dp_ref[1] + 1.0
    dinv = lax.rsqrt(deg)
    g_ref[...] = jnp.dot(x_ref[...], w_ref[...],
                         preferred_element_type=jnp.float32) * dinv
    dinv_ref[...] = dinv


def _make_tc1(interpret=False):
    return pl.pallas_call(
        _tc1_body,
        interpret=interpret,
        grid=(NBLK,),
        in_specs=[
            pl.BlockSpec((BLK, D), lambda b: (b, 0)),
            pl.BlockSpec((D, H), lambda b: (0, 0)),
            pl.BlockSpec((NC, BLK, H), lambda b: (0, b, 0)),
        ],
        out_specs=[pl.BlockSpec((BLK, H), lambda b: (b, 0)),
                   pl.BlockSpec((BLK, H), lambda b: (b, 0))],
        out_shape=[jax.ShapeDtypeStruct((NP, H), jnp.float32),
                   jax.ShapeDtypeStruct((NP, H), jnp.float32)],
    )


def _tc2_body(a_ref, g_ref, d_ref, b_ref, w_ref, o_ref):
    h1 = _elu(d_ref[...] * (a_ref[0] + a_ref[1] + g_ref[...]) + b_ref[...])
    o_ref[...] = jnp.dot(h1, w_ref[...],
                         preferred_element_type=jnp.float32) * d_ref[...]


def _make_tc2(interpret=False):
    return pl.pallas_call(
        _tc2_body,
        interpret=interpret,
        grid=(NBLK,),
        in_specs=[
            pl.BlockSpec((NC, BLK, H), lambda b: (0, b, 0)),
            pl.BlockSpec((BLK, H), lambda b: (b, 0)),
            pl.BlockSpec((BLK, H), lambda b: (b, 0)),
            pl.BlockSpec((1, H), lambda b: (0, 0)),
            pl.BlockSpec((H, H), lambda b: (0, 0)),
        ],
        out_specs=pl.BlockSpec((BLK, H), lambda b: (b, 0)),
        out_shape=jax.ShapeDtypeStruct((NP, H), jnp.float32),
    )


def _tc3_body(a_ref, g_ref, d_ref, b_ref, i_ref, w2_ref, c2_ref, w3_ref,
              c3_ref, o_ref, acc):
    pid = pl.program_id(0)
    h2 = _elu(d_ref[...] * (a_ref[0] + a_ref[1] + g_ref[...]) + b_ref[...])
    seg = i_ref[0]
    oh = (lax.broadcasted_iota(jnp.int32, (G, BLK), 0) == seg
          ).astype(jnp.float32)
    part = jnp.dot(oh, h2, preferred_element_type=jnp.float32)

    @pl.when(pid == 0)
    def _():
        acc[...] = part

    @pl.when(pid != 0)
    def _():
        acc[...] += part

    @pl.when(pid == NBLK - 1)
    def _():
        p = acc[...]
        t1 = jnp.maximum(
            jnp.dot(p, w2_ref[...], preferred_element_type=jnp.float32)
            + c2_ref[...], 0.0)
        z = jnp.dot(t1, w3_ref[...], preferred_element_type=jnp.float32) \
            + c3_ref[...]
        o_ref[...] = 1.0 / (1.0 + jnp.exp(-z))


def _make_tc3(interpret=False):
    return pl.pallas_call(
        _tc3_body,
        interpret=interpret,
        grid=(NBLK,),
        in_specs=[
            pl.BlockSpec((NC, BLK, H), lambda b: (0, b, 0)),
            pl.BlockSpec((BLK, H), lambda b: (b, 0)),
            pl.BlockSpec((BLK, H), lambda b: (b, 0)),
            pl.BlockSpec((1, H), lambda b: (0, 0)),
            pl.BlockSpec((1, 1, BLK), lambda b: (b, 0, 0)),
            pl.BlockSpec((H, H), lambda b: (0, 0)),
            pl.BlockSpec((1, H), lambda b: (0, 0)),
            pl.BlockSpec((H, O), lambda b: (0, 0)),
            pl.BlockSpec((1, O), lambda b: (0, 0)),
        ],
        out_specs=pl.BlockSpec((G, O), lambda b: (0, 0)),
        out_shape=jax.ShapeDtypeStruct((G, O), jnp.float32),
        scratch_shapes=[pltpu.VMEM((G, H), jnp.float32)],
    )


_tc1 = _make_tc1()
_tc2 = _make_tc2()
_tc3 = _make_tc3()


def kernel(x, edge_index, i, W1, b1, W2, b2, Wd2, bd2, Wd3, bd3):
    pad = jnp.full((2, EP - E), N, dtype=edge_index.dtype)
    e3 = jnp.concatenate([edge_index, pad], axis=1) \
            .reshape(2, NCHA, ECH).transpose(1, 0, 2)
    x_p = jnp.pad(x, ((0, NP - N), (0, 0)))
    i3 = jnp.pad(i, (0, NP - N), constant_values=G).reshape(NBLK, 1, BLK)

    sc_deg, sc_edges = _sc_kernels()
    degp = sc_deg(e3)
    g1, dinv = _tc1(x_p, W1, degp)
    agg1 = sc_edges(e3, g1)
    g2 = _tc2(agg1, g1, dinv, b1.reshape(1, H), W2)
    agg2 = sc_edges(e3, g2)
    out = _tc3(agg2, g2, dinv, b2.reshape(1, H), i3, Wd2,
               bd2.reshape(1, H), Wd3, bd3.reshape(1, O))
    return out
